# baseline (speedup 1.0000x reference)
import functools
import math

import jax
import jax.numpy as jnp
from jax import lax
from jax.experimental import pallas as pl
from jax.experimental.pallas import tpu as pltpu
from jax.experimental.pallas import tpu_sc as plsc

EMBED_DIM = 64
SCALE = math.sqrt(EMBED_DIM)

NUM_CORES = 2
NUM_SUBCORES = 16
NUM_WORKERS = NUM_CORES * NUM_SUBCORES
LANES = 16

BATCH, SEQ = 4096, 200
TOTAL = BATCH * SEQ
PER_WORKER = TOTAL // NUM_WORKERS
CHUNK = 400
N_CHUNKS = PER_WORKER // CHUNK


@functools.partial(
    pl.kernel,
    out_type=jax.ShapeDtypeStruct((TOTAL * EMBED_DIM,), jnp.float32),
    mesh=plsc.VectorSubcoreMesh(core_axis_name="c", subcore_axis_name="s"),
    compiler_params=pltpu.CompilerParams(
        use_tc_tiling_on_sc=False,
        skip_device_barrier=True,
        disable_bounds_checks=True,
        disable_semaphore_checks=True,
    ),
    scratch_types=[
        pltpu.VMEM((2, CHUNK), jnp.int32),
        pltpu.VMEM((2, CHUNK, EMBED_DIM), jnp.float32),
        pltpu.VMEM((2, CHUNK * EMBED_DIM), jnp.float32),
        pltpu.SemaphoreType.DMA,
        pltpu.SemaphoreType.DMA,
        pltpu.SemaphoreType.DMA,
        pltpu.SemaphoreType.DMA,
    ],
)
def _embed_lookup(tok_hbm, table_hbm, out_hbm, idx_v, rows_v, wb_v, g0, g1, o0, o1):
    wid = lax.axis_index("s") * NUM_CORES + lax.axis_index("c")
    base = wid * PER_WORKER
    gsem = (g0, g1)
    osem = (o0, o1)

    def idx_slice(g):
        return tok_hbm.at[pl.ds(base + g * CHUNK, CHUNK)]

    def out_slice(g):
        return out_hbm.at[pl.ds((base + g * CHUNK) * EMBED_DIM, CHUNK * EMBED_DIM)]

    def start_gather(s):
        return pltpu.async_copy(table_hbm.at[idx_v.at[s]], rows_v.at[s], gsem[s])

    def wait_gather(s):
        pltpu.make_async_copy(table_hbm.at[idx_v.at[s]], rows_v.at[s], gsem[s]).wait()

    def start_out(s, g):
        return pltpu.async_copy(wb_v.at[s], out_slice(g), osem[s])

    def wait_out(s, g):
        pltpu.make_async_copy(wb_v.at[s], out_slice(g), osem[s]).wait()

    def scale_chunk(s):
        def scale_row(j, _):
            for k in range(EMBED_DIM // LANES):
                sl = pl.ds(k * LANES, LANES)
                dsl = pl.ds(j * EMBED_DIM + k * LANES, LANES)
                wb_v[s, dsl] = rows_v[s, j, sl] * SCALE
            return 0

        lax.fori_loop(0, CHUNK, scale_row, 0, unroll=4)

    def steady(g, s):
        wait_out(1 - s, g - 1)
        start_gather(1 - s)
        wait_gather(s)
        pltpu.sync_copy(idx_slice(g + 2), idx_v.at[s])
        scale_chunk(s)
        start_out(s, g)

    pltpu.sync_copy(idx_slice(0), idx_v.at[0])
    start_gather(0)
    pltpu.sync_copy(idx_slice(1), idx_v.at[1])
    start_gather(1)
    wait_gather(0)
    pltpu.sync_copy(idx_slice(2), idx_v.at[0])
    scale_chunk(0)
    start_out(0, 0)
    wait_out(0, 0)
    start_gather(0)
    wait_gather(1)
    pltpu.sync_copy(idx_slice(3), idx_v.at[1])
    scale_chunk(1)
    start_out(1, 1)

    def pair_body(p, _):
        g = 2 * p
        steady(g, 0)
        steady(g + 1, 1)
        return 0

    lax.fori_loop(1, N_CHUNKS // 2 - 1, pair_body, 0)

    gl = N_CHUNKS - 2
    wait_out(1, gl - 1)
    start_gather(1)
    wait_gather(0)
    scale_chunk(0)
    start_out(0, gl)
    wait_gather(1)
    scale_chunk(1)
    start_out(1, gl + 1)
    wait_out(0, gl)
    wait_out(1, gl + 1)


def kernel(tokens, embedding):
    flat = jnp.reshape(tokens, (TOTAL,)).astype(jnp.int32)
    out = _embed_lookup(flat, embedding)
    return jnp.reshape(out, (BATCH, SEQ, EMBED_DIM))


# speedup vs baseline: 1.1567x; 1.1567x over previous
import functools
import math

import jax
import jax.numpy as jnp
from jax import lax
from jax.experimental import pallas as pl
from jax.experimental.pallas import tpu as pltpu
from jax.experimental.pallas import tpu_sc as plsc

EMBED_DIM = 64
SCALE = math.sqrt(EMBED_DIM)

NUM_CORES = 2
NUM_SUBCORES = 16
NUM_WORKERS = NUM_CORES * NUM_SUBCORES
LANES = 16

BATCH, SEQ = 4096, 200
TOTAL = BATCH * SEQ
PER_WORKER = TOTAL // NUM_WORKERS
CHUNK = 400
ROWS_PER_CHUNK = CHUNK // SEQ * 2
N_CHUNKS = PER_WORKER // CHUNK
ROWS_PER_WORKER = BATCH // NUM_WORKERS


@functools.partial(
    pl.kernel,
    out_type=jax.ShapeDtypeStruct((TOTAL, EMBED_DIM), jnp.float32),
    mesh=plsc.VectorSubcoreMesh(core_axis_name="c", subcore_axis_name="s"),
    compiler_params=pltpu.CompilerParams(
        use_tc_tiling_on_sc=False,
        skip_device_barrier=True,
        disable_bounds_checks=True,
        disable_semaphore_checks=True,
    ),
    scratch_types=[
        pltpu.VMEM((2, 2, SEQ), jnp.int32),
        pltpu.VMEM((2, CHUNK), jnp.int32),
        pltpu.VMEM((2, CHUNK, EMBED_DIM), jnp.float32),
        pltpu.SemaphoreType.DMA,
        pltpu.SemaphoreType.DMA,
        pltpu.SemaphoreType.DMA,
        pltpu.SemaphoreType.DMA,
    ],
)
def _embed_lookup(tok_hbm, table_hbm, out_hbm, tok_v, idx_v, rows_v, g0, g1, o0, o1):
    wid = lax.axis_index("s") * NUM_CORES + lax.axis_index("c")
    base = wid * PER_WORKER
    gsem = (g0, g1)
    osem = (o0, o1)

    row_base = wid * ROWS_PER_WORKER

    def load_idx(g, s):
        pltpu.sync_copy(
            tok_hbm.at[pl.ds(row_base + g * 2, 2)], tok_v.at[s]
        )

        def repack(i, _):
            idx_v[s, pl.ds(i * LANES, LANES)] = tok_v[
                s, i // (SEQ // 8), pl.ds((i % (SEQ // 8)) * LANES, LANES)
            ]
            return 0

        lax.fori_loop(0, CHUNK // LANES, repack, 0, unroll=5)

    def out_slice(g):
        return out_hbm.at[pl.ds(base + g * CHUNK, CHUNK)]

    def start_gather(s):
        return pltpu.async_copy(table_hbm.at[idx_v.at[s]], rows_v.at[s], gsem[s])

    def wait_gather(s):
        pltpu.make_async_copy(table_hbm.at[idx_v.at[s]], rows_v.at[s], gsem[s]).wait()

    def start_out(s, g):
        return pltpu.async_copy(rows_v.at[s], out_slice(g), osem[s])

    def wait_out(s, g):
        pltpu.make_async_copy(rows_v.at[s], out_slice(g), osem[s]).wait()

    def scale_chunk(s):
        def scale_row(j, _):
            for k in range(EMBED_DIM // LANES):
                sl = pl.ds(k * LANES, LANES)
                rows_v[s, j, sl] = rows_v[s, j, sl] * SCALE
            return 0

        lax.fori_loop(0, CHUNK, scale_row, 0, unroll=4)

    def steady(g, s):
        wait_out(1 - s, g - 1)
        start_gather(1 - s)
        wait_gather(s)
        load_idx(g + 2, s)
        scale_chunk(s)
        start_out(s, g)

    load_idx(0, 0)
    start_gather(0)
    load_idx(1, 1)
    start_gather(1)
    wait_gather(0)
    load_idx(2, 0)
    scale_chunk(0)
    start_out(0, 0)
    wait_out(0, 0)
    start_gather(0)
    wait_gather(1)
    load_idx(3, 1)
    scale_chunk(1)
    start_out(1, 1)

    def pair_body(p, _):
        g = 2 * p
        steady(g, 0)
        steady(g + 1, 1)
        return 0

    lax.fori_loop(1, N_CHUNKS // 2 - 1, pair_body, 0)

    gl = N_CHUNKS - 2
    wait_out(1, gl - 1)
    start_gather(1)
    wait_gather(0)
    scale_chunk(0)
    start_out(0, gl)
    wait_gather(1)
    scale_chunk(1)
    start_out(1, gl + 1)
    wait_out(0, gl)
    wait_out(1, gl + 1)


def kernel(tokens, embedding):
    out = _embed_lookup(tokens.astype(jnp.int32), embedding)
    return jnp.reshape(out, (BATCH, SEQ, EMBED_DIM))


# speedup vs baseline: 1.1739x; 1.0149x over previous
import functools
import math

import jax
import jax.numpy as jnp
from jax import lax
from jax.experimental import pallas as pl
from jax.experimental.pallas import tpu as pltpu
from jax.experimental.pallas import tpu_sc as plsc

EMBED_DIM = 64
SCALE = math.sqrt(EMBED_DIM)

NUM_CORES = 2
NUM_SUBCORES = 16
NUM_WORKERS = NUM_CORES * NUM_SUBCORES
LANES = 16

BATCH, SEQ = 4096, 200
TOTAL = BATCH * SEQ
PER_WORKER = TOTAL // NUM_WORKERS
CHUNK = 800
N_CHUNKS = PER_WORKER // CHUNK


@functools.partial(
    pl.kernel,
    out_type=jax.ShapeDtypeStruct((TOTAL, EMBED_DIM), jnp.float32),
    mesh=plsc.VectorSubcoreMesh(core_axis_name="c", subcore_axis_name="s"),
    compiler_params=pltpu.CompilerParams(
        use_tc_tiling_on_sc=False,
        skip_device_barrier=True,
        disable_bounds_checks=True,
        disable_semaphore_checks=True,
    ),
    scratch_types=[
        pltpu.VMEM((2, CHUNK), jnp.int32),
        pltpu.VMEM((2, CHUNK, EMBED_DIM), jnp.float32),
        pltpu.SemaphoreType.DMA,
        pltpu.SemaphoreType.DMA,
        pltpu.SemaphoreType.DMA,
        pltpu.SemaphoreType.DMA,
    ],
)
def _embed_lookup(tok_hbm, table_hbm, out_hbm, idx_v, rows_v, g0, g1, o0, o1):
    wid = lax.axis_index("s") * NUM_CORES + lax.axis_index("c")
    base = wid * PER_WORKER
    gsem = (g0, g1)
    osem = (o0, o1)

    def idx_slice(g):
        return tok_hbm.at[pl.ds(base + g * CHUNK, CHUNK)]

    def out_slice(g):
        return out_hbm.at[pl.ds(base + g * CHUNK, CHUNK)]

    def start_gather(s):
        return pltpu.async_copy(table_hbm.at[idx_v.at[s]], rows_v.at[s], gsem[s])

    def wait_gather(s):
        pltpu.make_async_copy(table_hbm.at[idx_v.at[s]], rows_v.at[s], gsem[s]).wait()

    def start_out(s, g):
        return pltpu.async_copy(rows_v.at[s], out_slice(g), osem[s])

    def wait_out(s, g):
        pltpu.make_async_copy(rows_v.at[s], out_slice(g), osem[s]).wait()

    def scale_chunk(s):
        def scale_row(j, _):
            for k in range(EMBED_DIM // LANES):
                sl = pl.ds(k * LANES, LANES)
                rows_v[s, j, sl] = rows_v[s, j, sl] * SCALE
            return 0

        lax.fori_loop(0, CHUNK, scale_row, 0, unroll=4)

    def steady(g, s):
        wait_out(1 - s, g - 1)
        start_gather(1 - s)
        wait_gather(s)
        pltpu.sync_copy(idx_slice(g + 2), idx_v.at[s])
        scale_chunk(s)
        start_out(s, g)

    pltpu.sync_copy(idx_slice(0), idx_v.at[0])
    start_gather(0)
    pltpu.sync_copy(idx_slice(1), idx_v.at[1])
    start_gather(1)
    wait_gather(0)
    pltpu.sync_copy(idx_slice(2), idx_v.at[0])
    scale_chunk(0)
    start_out(0, 0)
    wait_out(0, 0)
    start_gather(0)
    wait_gather(1)
    pltpu.sync_copy(idx_slice(3), idx_v.at[1])
    scale_chunk(1)
    start_out(1, 1)

    def pair_body(p, _):
        g = 2 * p
        steady(g, 0)
        steady(g + 1, 1)
        return 0

    lax.fori_loop(1, N_CHUNKS // 2 - 1, pair_body, 0)

    gl = N_CHUNKS - 2
    wait_out(1, gl - 1)
    start_gather(1)
    wait_gather(0)
    scale_chunk(0)
    start_out(0, gl)
    wait_gather(1)
    scale_chunk(1)
    start_out(1, gl + 1)
    wait_out(0, gl)
    wait_out(1, gl + 1)


def kernel(tokens, embedding):
    flat = jnp.reshape(tokens, (TOTAL,)).astype(jnp.int32)
    out = _embed_lookup(flat, embedding)
    return jnp.reshape(out, (BATCH, SEQ, EMBED_DIM))


# speedup vs baseline: 1.1746x; 1.0006x over previous
import functools
import math

import jax
import jax.numpy as jnp
from jax import lax
from jax.experimental import pallas as pl
from jax.experimental.pallas import tpu as pltpu
from jax.experimental.pallas import tpu_sc as plsc

EMBED_DIM = 64
SCALE = math.sqrt(EMBED_DIM)

NUM_CORES = 2
NUM_SUBCORES = 16
NUM_WORKERS = NUM_CORES * NUM_SUBCORES
LANES = 16

BATCH, SEQ = 4096, 200
TOTAL = BATCH * SEQ
PER_WORKER = TOTAL // NUM_WORKERS
CHUNK = 800
N_CHUNKS = PER_WORKER // CHUNK


@functools.partial(
    pl.kernel,
    out_type=jax.ShapeDtypeStruct((TOTAL, EMBED_DIM), jnp.float32),
    mesh=plsc.VectorSubcoreMesh(
        core_axis_name="c",
        subcore_axis_name="s",
        num_cores=NUM_CORES,
        num_subcores=NUM_SUBCORES,
    ),
    compiler_params=pltpu.CompilerParams(
        use_tc_tiling_on_sc=False,
        skip_device_barrier=True,
        disable_bounds_checks=True,
        disable_semaphore_checks=True,
    ),
    scratch_types=[
        pltpu.VMEM((2, CHUNK), jnp.int32),
        pltpu.VMEM((2, CHUNK, EMBED_DIM), jnp.float32),
        pltpu.SemaphoreType.DMA,
        pltpu.SemaphoreType.DMA,
        pltpu.SemaphoreType.DMA,
        pltpu.SemaphoreType.DMA,
    ],
)
def _embed_lookup(tok_hbm, table_hbm, out_hbm, idx_v, rows_v, g0, g1, o0, o1):
    wid = lax.axis_index("s") * NUM_CORES + lax.axis_index("c")
    base = wid * PER_WORKER
    gsem = (g0, g1)
    osem = (o0, o1)

    def idx_slice(g):
        return tok_hbm.at[pl.ds(base + g * CHUNK, CHUNK)]

    def out_slice(g):
        return out_hbm.at[pl.ds(base + g * CHUNK, CHUNK)]

    def start_gather(s):
        return pltpu.async_copy(table_hbm.at[idx_v.at[s]], rows_v.at[s], gsem[s])

    def wait_gather(s):
        pltpu.make_async_copy(table_hbm.at[idx_v.at[s]], rows_v.at[s], gsem[s]).wait()

    def start_out(s, g):
        return pltpu.async_copy(rows_v.at[s], out_slice(g), osem[s])

    def wait_out(s, g):
        pltpu.make_async_copy(rows_v.at[s], out_slice(g), osem[s]).wait()

    def scale_chunk(s):
        def scale_row(j, _):
            for k in range(EMBED_DIM // LANES):
                sl = pl.ds(k * LANES, LANES)
                rows_v[s, j, sl] = rows_v[s, j, sl] * SCALE
            return 0

        lax.fori_loop(0, CHUNK, scale_row, 0, unroll=4)

    def steady(g, s):
        wait_out(1 - s, g - 1)
        start_gather(1 - s)
        wait_gather(s)
        pltpu.sync_copy(idx_slice(g + 2), idx_v.at[s])
        scale_chunk(s)
        start_out(s, g)

    pltpu.sync_copy(idx_slice(0), idx_v.at[0])
    start_gather(0)
    pltpu.sync_copy(idx_slice(1), idx_v.at[1])
    start_gather(1)
    wait_gather(0)
    pltpu.sync_copy(idx_slice(2), idx_v.at[0])
    scale_chunk(0)
    start_out(0, 0)
    wait_out(0, 0)
    start_gather(0)
    wait_gather(1)
    pltpu.sync_copy(idx_slice(3), idx_v.at[1])
    scale_chunk(1)
    start_out(1, 1)

    def pair_body(p, _):
        g = 2 * p
        steady(g, 0)
        steady(g + 1, 1)
        return 0

    lax.fori_loop(1, N_CHUNKS // 2 - 1, pair_body, 0)

    gl = N_CHUNKS - 2
    wait_out(1, gl - 1)
    start_gather(1)
    wait_gather(0)
    scale_chunk(0)
    start_out(0, gl)
    wait_gather(1)
    scale_chunk(1)
    start_out(1, gl + 1)
    wait_out(0, gl)
    wait_out(1, gl + 1)


def kernel(tokens, embedding):
    flat = jnp.reshape(tokens, (TOTAL,)).astype(jnp.int32)
    out = _embed_lookup(flat, embedding)
    return jnp.reshape(out, (BATCH, SEQ, EMBED_DIM))
